# baseline (speedup 1.0000x reference)
import functools

import jax
import jax.numpy as jnp
from jax.experimental import pallas as pl
from jax.experimental.pallas import tpu as pltpu

_AVG_D_LOG = 3.4965075614664802


def _node_pre_kernel(h_ref, p_ref, wh_ref, wp_ref, ah_ref, bh_ref, ap_ref, bp_ref):
    abh = jnp.dot(h_ref[...], wh_ref[...], preferred_element_type=jnp.float32)
    abp = jnp.dot(p_ref[...], wp_ref[...], preferred_element_type=jnp.float32)
    ah_ref[...] = abh[:, :128]
    bh_ref[...] = abh[:, 128:]
    ap_ref[...] = abp[:, :128]
    bp_ref[...] = abp[:, 128:]


def _node_pre(h, p, wh, wp, blk=1000):
    n = h.shape[0]
    grid = n // blk
    out = jax.ShapeDtypeStruct((n, 128), jnp.float32)
    return pl.pallas_call(
        _node_pre_kernel,
        grid=(grid,),
        in_specs=[
            pl.BlockSpec((blk, h.shape[1]), lambda i: (i, 0)),
            pl.BlockSpec((blk, p.shape[1]), lambda i: (i, 0)),
            pl.BlockSpec(wh.shape, lambda i: (0, 0)),
            pl.BlockSpec(wp.shape, lambda i: (0, 0)),
        ],
        out_specs=[pl.BlockSpec((blk, 128), lambda i: (i, 0))] * 4,
        out_shape=[out, out, out, out],
    )(h, p, wh, wp)


def _edge_mlp_kernel(uh_ref, up_ref, e_ref, weh_ref, wep_ref, b1h_ref, b1p_ref,
                     w2h_ref, w2p_ref, b2h_ref, b2p_ref, mh_ref, mp_ref):
    e = e_ref[...]
    th = uh_ref[...] + jnp.dot(e, weh_ref[...], preferred_element_type=jnp.float32) + b1h_ref[...]
    th = jax.nn.relu(th)
    mh_ref[...] = jnp.dot(th, w2h_ref[...], preferred_element_type=jnp.float32) + b2h_ref[...]
    tp = up_ref[...] + jnp.dot(e, wep_ref[...], preferred_element_type=jnp.float32) + b1p_ref[...]
    tp = jnp.tanh(tp)
    mp_ref[...] = jnp.dot(tp, w2p_ref[...], preferred_element_type=jnp.float32) + b2p_ref[...]


def _edge_mlp(uh, up, e, weh, wep, b1h, b1p, w2h, w2p, b2h, b2p, blk=2000):
    ne = e.shape[0]
    grid = ne // blk
    out = jax.ShapeDtypeStruct((ne, 128), jnp.float32)
    full = lambda a: pl.BlockSpec(a.shape, lambda i: (0,) * a.ndim)
    return pl.pallas_call(
        _edge_mlp_kernel,
        grid=(grid,),
        in_specs=[
            pl.BlockSpec((blk, 128), lambda i: (i, 0)),
            pl.BlockSpec((blk, 128), lambda i: (i, 0)),
            pl.BlockSpec((blk, e.shape[1]), lambda i: (i, 0)),
            full(weh), full(wep), full(b1h), full(b1p),
            full(w2h), full(w2p), full(b2h), full(b2p),
        ],
        out_specs=[pl.BlockSpec((blk, 128), lambda i: (i, 0))] * 2,
        out_shape=[out, out],
    )(uh, up, e, weh, wep, b1h, b1p, w2h, w2p, b2h, b2p)


def _post_kernel(h_ref, p_ref, snorm_ref, deg_ref,
                 sh_ref, qh_ref, xh_ref, nh_ref,
                 sp_ref, qp_ref, xp_ref, np_ref,
                 w1hh_ref, w1ha_ref, w1hb_ref, w1hc_ref, b1h_ref, w2h_ref, b2h_ref,
                 w1pp_ref, w1pa_ref, w1pb_ref, w1pc_ref, b1p_ref, w2p_ref, b2p_ref,
                 hpre_ref, pout_ref, psum_ref, psq_ref):
    deg = deg_ref[...]
    degc = jnp.maximum(deg, 1.0)
    present = deg > 0.0

    def agg_of(s, q, mx, mn):
        mean = s / degc
        mx = jnp.where(present, mx, 0.0)
        mn = jnp.where(present, mn, 0.0)
        var = jax.nn.relu(q / degc - mean * mean)
        std = jnp.sqrt(var + 1e-5)
        return jnp.concatenate([mean, mx, mn, std], axis=1)

    agg_h = agg_of(sh_ref[...], qh_ref[...], xh_ref[...], nh_ref[...])
    agg_p = agg_of(sp_ref[...], qp_ref[...], xp_ref[...], np_ref[...])
    logd = jnp.log(degc[:, :1] + 1.0)
    s1 = logd / _AVG_D_LOG
    s2 = _AVG_D_LOG / logd

    def post(xself, w_self, agg, wa, wb, wc, b1, w2, b2, act):
        y = jnp.dot(xself, w_self[...], preferred_element_type=jnp.float32)
        y += jnp.dot(agg, wa[...], preferred_element_type=jnp.float32)
        y += s1 * jnp.dot(agg, wb[...], preferred_element_type=jnp.float32)
        y += s2 * jnp.dot(agg, wc[...], preferred_element_type=jnp.float32)
        y = act(y + b1[...])
        return jnp.dot(y, w2[...], preferred_element_type=jnp.float32) + b2[...]

    hp = post(h_ref[...], w1hh_ref, agg_h, w1ha_ref, w1hb_ref, w1hc_ref,
              b1h_ref, w2h_ref, b2h_ref, jax.nn.relu)
    hp = hp * snorm_ref[...]
    hpre_ref[...] = hp
    pout_ref[...] = post(p_ref[...], w1pp_ref, agg_p, w1pa_ref, w1pb_ref, w1pc_ref,
                         b1p_ref, w2p_ref, b2p_ref, jnp.tanh)
    psum_ref[...] = jnp.sum(hp, axis=0, keepdims=True)[None]
    psq_ref[...] = jnp.sum(hp * hp, axis=0, keepdims=True)[None]


def _post(h, p, snorm, deg, aggs, wpost, blk=1000):
    n = h.shape[0]
    grid = n // blk
    sh, qh, xh, nh, sp, qp, xp, np_ = aggs
    full = lambda a: pl.BlockSpec(a.shape, lambda i: (0,) * a.ndim)
    row = lambda w: pl.BlockSpec((blk, w), lambda i: (i, 0))
    return pl.pallas_call(
        _post_kernel,
        grid=(grid,),
        in_specs=[row(h.shape[1]), row(p.shape[1]), row(1), row(128)]
                 + [row(128)] * 8
                 + [full(w) for w in wpost],
        out_specs=[row(128), row(128),
                   pl.BlockSpec((1, 1, 128), lambda i: (i, 0, 0)),
                   pl.BlockSpec((1, 1, 128), lambda i: (i, 0, 0))],
        out_shape=[jax.ShapeDtypeStruct((n, 128), jnp.float32),
                   jax.ShapeDtypeStruct((n, 128), jnp.float32),
                   jax.ShapeDtypeStruct((grid, 1, 128), jnp.float32),
                   jax.ShapeDtypeStruct((grid, 1, 128), jnp.float32)],
    )(h, p, snorm, deg, sh, qh, xh, nh, sp, qp, xp, np_, *wpost)


def _bn_kernel(hpre_ref, psum_ref, psq_ref, g_ref, b_ref, o_ref, *, n_total):
    mu = jnp.sum(psum_ref[...], axis=(0, 1), keepdims=False)[None] / n_total
    var = jnp.sum(psq_ref[...], axis=(0, 1), keepdims=False)[None] / n_total - mu * mu
    inv = jax.lax.rsqrt(var + 1e-5)
    o_ref[...] = (hpre_ref[...] - mu) * inv * g_ref[...] + b_ref[...]


def _bn(hpre, psum, psq, gamma, beta, blk=1000):
    n = hpre.shape[0]
    grid = n // blk
    full = lambda a: pl.BlockSpec(a.shape, lambda i: (0,) * a.ndim)
    return pl.pallas_call(
        functools.partial(_bn_kernel, n_total=float(n)),
        grid=(grid,),
        in_specs=[pl.BlockSpec((blk, 128), lambda i: (i, 0)),
                  full(psum), full(psq), full(gamma), full(beta)],
        out_specs=pl.BlockSpec((blk, 128), lambda i: (i, 0)),
        out_shape=jax.ShapeDtypeStruct((n, 128), jnp.float32),
    )(hpre, psum, psq, gamma, beta)


def kernel(h, p, e, snorm_n, edge_index,
           pre_h_W1, pre_h_b1, pre_h_W2, pre_h_b2,
           pre_p_W1, pre_p_b1, pre_p_W2, pre_p_b2,
           post_h_W1, post_h_b1, post_h_W2, post_h_b2,
           post_p_W1, post_p_b1, post_p_W2, post_p_b2,
           bn_gamma, bn_beta):
    n = h.shape[0]
    ne = e.shape[0]
    src = edge_index[0]
    dst = edge_index[1]

    f = 128
    wh_node = jnp.concatenate([pre_h_W1[:2 * f], pre_h_W1[2 * f:4 * f]], axis=1)
    wp_node = jnp.concatenate([pre_p_W1[:f], pre_p_W1[f:2 * f]], axis=1)
    weh = pre_h_W1[4 * f:]
    wep = pre_p_W1[2 * f:]
    b1h = pre_h_b1.reshape(1, -1)
    b1p = pre_p_b1.reshape(1, -1)
    b2h = pre_h_b2.reshape(1, -1)
    b2p = pre_p_b2.reshape(1, -1)

    ah, bh, ap, bp = _node_pre(h, p, wh_node, wp_node)

    uh = ah[src] + bh[dst]
    up = ap[src] + bp[dst]

    mh, mp = _edge_mlp(uh, up, e, weh, wep, b1h, b1p,
                       pre_h_W2, pre_p_W2, b2h, b2p)

    ones = jnp.ones((ne,), jnp.float32)
    deg = jax.ops.segment_sum(ones, dst, num_segments=n)
    sh = jax.ops.segment_sum(mh, dst, num_segments=n)
    qh = jax.ops.segment_sum(mh * mh, dst, num_segments=n)
    xh = jax.ops.segment_max(mh, dst, num_segments=n)
    nh = jax.ops.segment_min(mh, dst, num_segments=n)
    sp = jax.ops.segment_sum(mp, dst, num_segments=n)
    qp = jax.ops.segment_sum(mp * mp, dst, num_segments=n)
    xp = jax.ops.segment_max(mp, dst, num_segments=n)
    np_ = jax.ops.segment_min(mp, dst, num_segments=n)
    xh = jnp.where(deg[:, None] > 0, xh, -3.0e38)
    nh = jnp.where(deg[:, None] > 0, nh, 3.0e38)
    xp = jnp.where(deg[:, None] > 0, xp, -3.0e38)
    np_ = jnp.where(deg[:, None] > 0, np_, 3.0e38)
    degb = jnp.broadcast_to(deg[:, None], (n, 128))

    wpost = [post_h_W1[:2 * f], post_h_W1[2 * f:6 * f],
             post_h_W1[6 * f:10 * f], post_h_W1[10 * f:14 * f],
             post_h_b1.reshape(1, -1), post_h_W2, post_h_b2.reshape(1, -1),
             post_p_W1[:f], post_p_W1[f:5 * f],
             post_p_W1[5 * f:9 * f], post_p_W1[9 * f:13 * f],
             post_p_b1.reshape(1, -1), post_p_W2, post_p_b2.reshape(1, -1)]

    hpre, pout, psum, psq = _post(h, p, snorm_n, degb,
                                  (sh, qh, xh, nh, sp, qp, xp, np_), wpost)
    hout = _bn(hpre, psum, psq, bn_gamma.reshape(1, -1), bn_beta.reshape(1, -1))
    return (hout, pout)


# speedup vs baseline: 1.3553x; 1.3553x over previous
import functools

import jax
import jax.numpy as jnp
from jax import lax
from jax.experimental import pallas as pl
from jax.experimental.pallas import tpu as pltpu
from jax.experimental.pallas import tpu_sc as plsc

_NC, _NS = 2, 16
_NW = _NC * _NS

_AVG_D_LOG = 3.4965075614664802


def _node_pre_kernel(h_ref, p_ref, wh_ref, wp_ref, ah_ref, bh_ref, ap_ref, bp_ref):
    abh = jnp.dot(h_ref[...], wh_ref[...], preferred_element_type=jnp.float32)
    abp = jnp.dot(p_ref[...], wp_ref[...], preferred_element_type=jnp.float32)
    ah_ref[...] = abh[:, :128]
    bh_ref[...] = abh[:, 128:]
    ap_ref[...] = abp[:, :128]
    bp_ref[...] = abp[:, 128:]


def _node_pre(h, p, wh, wp, blk=1000):
    n = h.shape[0]
    grid = n // blk
    out = jax.ShapeDtypeStruct((n, 128), jnp.float32)
    return pl.pallas_call(
        _node_pre_kernel,
        grid=(grid,),
        in_specs=[
            pl.BlockSpec((blk, h.shape[1]), lambda i: (i, 0)),
            pl.BlockSpec((blk, p.shape[1]), lambda i: (i, 0)),
            pl.BlockSpec(wh.shape, lambda i: (0, 0)),
            pl.BlockSpec(wp.shape, lambda i: (0, 0)),
        ],
        out_specs=[pl.BlockSpec((blk, 128), lambda i: (i, 0))] * 4,
        out_shape=[out, out, out, out],
    )(h, p, wh, wp)


def _edge_mlp_kernel(uh_ref, up_ref, e_ref, weh_ref, wep_ref, b1h_ref, b1p_ref,
                     w2h_ref, w2p_ref, b2h_ref, b2p_ref, mh_ref, mp_ref):
    e = e_ref[...]
    th = uh_ref[...] + jnp.dot(e, weh_ref[...], preferred_element_type=jnp.float32) + b1h_ref[...]
    th = jax.nn.relu(th)
    mh_ref[...] = jnp.dot(th, w2h_ref[...], preferred_element_type=jnp.float32) + b2h_ref[...]
    tp = up_ref[...] + jnp.dot(e, wep_ref[...], preferred_element_type=jnp.float32) + b1p_ref[...]
    tp = jnp.tanh(tp)
    mp_ref[...] = jnp.dot(tp, w2p_ref[...], preferred_element_type=jnp.float32) + b2p_ref[...]


def _edge_mlp(uh, up, e, weh, wep, b1h, b1p, w2h, w2p, b2h, b2p, blk=2000):
    ne = e.shape[0]
    grid = ne // blk
    out = jax.ShapeDtypeStruct((ne, 128), jnp.float32)
    full = lambda a: pl.BlockSpec(a.shape, lambda i: (0,) * a.ndim)
    return pl.pallas_call(
        _edge_mlp_kernel,
        grid=(grid,),
        in_specs=[
            pl.BlockSpec((blk, 128), lambda i: (i, 0)),
            pl.BlockSpec((blk, 128), lambda i: (i, 0)),
            pl.BlockSpec((blk, e.shape[1]), lambda i: (i, 0)),
            full(weh), full(wep), full(b1h), full(b1p),
            full(w2h), full(w2p), full(b2h), full(b2p),
        ],
        out_specs=[pl.BlockSpec((blk, 128), lambda i: (i, 0))] * 2,
        out_shape=[out, out],
    )(uh, up, e, weh, wep, b1h, b1p, w2h, w2p, b2h, b2p)


def _post_kernel(h_ref, p_ref, snorm_ref, deg_ref,
                 sh_ref, qh_ref, xh_ref, nh_ref,
                 sp_ref, qp_ref, xp_ref, np_ref,
                 w1hh_ref, w1ha_ref, w1hb_ref, w1hc_ref, b1h_ref, w2h_ref, b2h_ref,
                 w1pp_ref, w1pa_ref, w1pb_ref, w1pc_ref, b1p_ref, w2p_ref, b2p_ref,
                 hpre_ref, pout_ref, psum_ref, psq_ref):
    deg = deg_ref[...]
    degc = jnp.maximum(deg, 1.0)
    present = deg > 0.0

    def agg_of(s, q, mx, mn):
        mean = s / degc
        mx = jnp.where(present, mx, 0.0)
        mn = jnp.where(present, mn, 0.0)
        var = jax.nn.relu(q / degc - mean * mean)
        std = jnp.sqrt(var + 1e-5)
        return jnp.concatenate([mean, mx, mn, std], axis=1)

    agg_h = agg_of(sh_ref[...], qh_ref[...], xh_ref[...], nh_ref[...])
    agg_p = agg_of(sp_ref[...], qp_ref[...], xp_ref[...], np_ref[...])
    logd = jnp.log(degc[:, :1] + 1.0)
    s1 = logd / _AVG_D_LOG
    s2 = _AVG_D_LOG / logd

    def post(xself, w_self, agg, wa, wb, wc, b1, w2, b2, act):
        y = jnp.dot(xself, w_self[...], preferred_element_type=jnp.float32)
        y += jnp.dot(agg, wa[...], preferred_element_type=jnp.float32)
        y += s1 * jnp.dot(agg, wb[...], preferred_element_type=jnp.float32)
        y += s2 * jnp.dot(agg, wc[...], preferred_element_type=jnp.float32)
        y = act(y + b1[...])
        return jnp.dot(y, w2[...], preferred_element_type=jnp.float32) + b2[...]

    hp = post(h_ref[...], w1hh_ref, agg_h, w1ha_ref, w1hb_ref, w1hc_ref,
              b1h_ref, w2h_ref, b2h_ref, jax.nn.relu)
    hp = hp * snorm_ref[...]
    hpre_ref[...] = hp
    pout_ref[...] = post(p_ref[...], w1pp_ref, agg_p, w1pa_ref, w1pb_ref, w1pc_ref,
                         b1p_ref, w2p_ref, b2p_ref, jnp.tanh)
    psum_ref[...] = jnp.sum(hp, axis=0, keepdims=True)[None]
    psq_ref[...] = jnp.sum(hp * hp, axis=0, keepdims=True)[None]


def _post(h, p, snorm, deg, aggs, wpost, blk=1000):
    n = h.shape[0]
    grid = n // blk
    sh, qh, xh, nh, sp, qp, xp, np_ = aggs
    full = lambda a: pl.BlockSpec(a.shape, lambda i: (0,) * a.ndim)
    row = lambda w: pl.BlockSpec((blk, w), lambda i: (i, 0))
    return pl.pallas_call(
        _post_kernel,
        grid=(grid,),
        in_specs=[row(h.shape[1]), row(p.shape[1]), row(1), row(128)]
                 + [row(128)] * 8
                 + [full(w) for w in wpost],
        out_specs=[row(128), row(128),
                   pl.BlockSpec((1, 1, 128), lambda i: (i, 0, 0)),
                   pl.BlockSpec((1, 1, 128), lambda i: (i, 0, 0))],
        out_shape=[jax.ShapeDtypeStruct((n, 128), jnp.float32),
                   jax.ShapeDtypeStruct((n, 128), jnp.float32),
                   jax.ShapeDtypeStruct((grid, 1, 128), jnp.float32),
                   jax.ShapeDtypeStruct((grid, 1, 128), jnp.float32)],
    )(h, p, snorm, deg, sh, qh, xh, nh, sp, qp, xp, np_, *wpost)


def _bn_kernel(hpre_ref, psum_ref, psq_ref, g_ref, b_ref, o_ref, *, n_total):
    mu = jnp.sum(psum_ref[...], axis=(0, 1), keepdims=False)[None] / n_total
    var = jnp.sum(psq_ref[...], axis=(0, 1), keepdims=False)[None] / n_total - mu * mu
    inv = jax.lax.rsqrt(var + 1e-5)
    o_ref[...] = (hpre_ref[...] - mu) * inv * g_ref[...] + b_ref[...]


def _bn(hpre, psum, psq, gamma, beta, blk=1000):
    n = hpre.shape[0]
    grid = n // blk
    full = lambda a: pl.BlockSpec(a.shape, lambda i: (0,) * a.ndim)
    return pl.pallas_call(
        functools.partial(_bn_kernel, n_total=float(n)),
        grid=(grid,),
        in_specs=[pl.BlockSpec((blk, 128), lambda i: (i, 0)),
                  full(psum), full(psq), full(gamma), full(beta)],
        out_specs=pl.BlockSpec((blk, 128), lambda i: (i, 0)),
        out_shape=jax.ShapeDtypeStruct((n, 128), jnp.float32),
    )(hpre, psum, psq, gamma, beta)


def _sc_gather(ah, bh, ap, bp, src, dst):
    n_e = src.shape[0]
    per_w = n_e // _NW
    w_blk = 200
    n_win = per_w // w_blk
    mesh = plsc.VectorSubcoreMesh(core_axis_name="c", subcore_axis_name="s")
    out = jax.ShapeDtypeStruct((n_e, 128), jnp.float32)

    @functools.partial(
        pl.kernel, mesh=mesh,
        out_type=[out, out],
        scratch_types=[
            pltpu.VMEM((w_blk,), jnp.int32),
            pltpu.VMEM((w_blk,), jnp.int32),
            pltpu.VMEM((w_blk, 128), jnp.float32),
            pltpu.VMEM((w_blk, 128), jnp.float32),
            pltpu.SemaphoreType.DMA,
        ],
    )
    def k(ah_h, bh_h, ap_h, bp_h, src_h, dst_h, uh_h, up_h,
          si_v, di_v, ra_v, rb_v, sem):
        wid = lax.axis_index("s") * _NC + lax.axis_index("c")
        base0 = wid * per_w

        def do_path(a_h, b_h, u_h):
            def body(w, _):
                base = base0 + w * w_blk
                pltpu.sync_copy(src_h.at[pl.ds(base, w_blk)], si_v)
                pltpu.sync_copy(dst_h.at[pl.ds(base, w_blk)], di_v)
                ca = pltpu.async_copy(a_h.at[si_v], ra_v, sem)
                cb = pltpu.async_copy(b_h.at[di_v], rb_v, sem)
                ca.wait()
                cb.wait()

                def add_row(i, _):
                    for j in range(8):
                        sl = pl.ds(j * 16, 16)
                        ra_v[i, sl] = ra_v[i, sl] + rb_v[i, sl]
                    return 0
                lax.fori_loop(0, w_blk, add_row, 0)
                pltpu.sync_copy(ra_v, u_h.at[pl.ds(base, w_blk)])
                return 0
            lax.fori_loop(0, n_win, body, 0)

        do_path(ah_h, bh_h, uh_h)
        do_path(ap_h, bp_h, up_h)

    return k(ah, bh, ap, bp, src, dst)


def kernel(h, p, e, snorm_n, edge_index,
           pre_h_W1, pre_h_b1, pre_h_W2, pre_h_b2,
           pre_p_W1, pre_p_b1, pre_p_W2, pre_p_b2,
           post_h_W1, post_h_b1, post_h_W2, post_h_b2,
           post_p_W1, post_p_b1, post_p_W2, post_p_b2,
           bn_gamma, bn_beta):
    n = h.shape[0]
    ne = e.shape[0]
    src = edge_index[0]
    dst = edge_index[1]

    f = 128
    wh_node = jnp.concatenate([pre_h_W1[:2 * f], pre_h_W1[2 * f:4 * f]], axis=1)
    wp_node = jnp.concatenate([pre_p_W1[:f], pre_p_W1[f:2 * f]], axis=1)
    weh = pre_h_W1[4 * f:]
    wep = pre_p_W1[2 * f:]
    b1h = pre_h_b1.reshape(1, -1)
    b1p = pre_p_b1.reshape(1, -1)
    b2h = pre_h_b2.reshape(1, -1)
    b2p = pre_p_b2.reshape(1, -1)

    ah, bh, ap, bp = _node_pre(h, p, wh_node, wp_node)

    uh, up = _sc_gather(ah, bh, ap, bp, src, dst)

    mh, mp = _edge_mlp(uh, up, e, weh, wep, b1h, b1p,
                       pre_h_W2, pre_p_W2, b2h, b2p)

    ones = jnp.ones((ne,), jnp.float32)
    deg = jax.ops.segment_sum(ones, dst, num_segments=n)
    sh = jax.ops.segment_sum(mh, dst, num_segments=n)
    qh = jax.ops.segment_sum(mh * mh, dst, num_segments=n)
    xh = jax.ops.segment_max(mh, dst, num_segments=n)
    nh = jax.ops.segment_min(mh, dst, num_segments=n)
    sp = jax.ops.segment_sum(mp, dst, num_segments=n)
    qp = jax.ops.segment_sum(mp * mp, dst, num_segments=n)
    xp = jax.ops.segment_max(mp, dst, num_segments=n)
    np_ = jax.ops.segment_min(mp, dst, num_segments=n)
    xh = jnp.where(deg[:, None] > 0, xh, -3.0e38)
    nh = jnp.where(deg[:, None] > 0, nh, 3.0e38)
    xp = jnp.where(deg[:, None] > 0, xp, -3.0e38)
    np_ = jnp.where(deg[:, None] > 0, np_, 3.0e38)
    degb = jnp.broadcast_to(deg[:, None], (n, 128))

    wpost = [post_h_W1[:2 * f], post_h_W1[2 * f:6 * f],
             post_h_W1[6 * f:10 * f], post_h_W1[10 * f:14 * f],
             post_h_b1.reshape(1, -1), post_h_W2, post_h_b2.reshape(1, -1),
             post_p_W1[:f], post_p_W1[f:5 * f],
             post_p_W1[5 * f:9 * f], post_p_W1[9 * f:13 * f],
             post_p_b1.reshape(1, -1), post_p_W2, post_p_b2.reshape(1, -1)]

    hpre, pout, psum, psq = _post(h, p, snorm_n, degb,
                                  (sh, qh, xh, nh, sp, qp, xp, np_), wpost)
    hout = _bn(hpre, psum, psq, bn_gamma.reshape(1, -1), bn_beta.reshape(1, -1))
    return (hout, pout)


# speedup vs baseline: 2.1947x; 1.6194x over previous
import functools

import jax
import jax.numpy as jnp
from jax import lax
from jax.experimental import pallas as pl
from jax.experimental.pallas import tpu as pltpu
from jax.experimental.pallas import tpu_sc as plsc

_NC, _NS = 2, 16
_NW = _NC * _NS

_AVG_D_LOG = 3.4965075614664802


def _node_pre_kernel(h_ref, p_ref, wh_ref, wp_ref, ah_ref, bh_ref, ap_ref, bp_ref):
    abh = jnp.dot(h_ref[...], wh_ref[...], preferred_element_type=jnp.float32)
    abp = jnp.dot(p_ref[...], wp_ref[...], preferred_element_type=jnp.float32)
    ah_ref[...] = abh[:, :128]
    bh_ref[...] = abh[:, 128:]
    ap_ref[...] = abp[:, :128]
    bp_ref[...] = abp[:, 128:]


def _node_pre(h, p, wh, wp, blk=1000):
    n = h.shape[0]
    grid = n // blk
    out = jax.ShapeDtypeStruct((n, 128), jnp.float32)
    return pl.pallas_call(
        _node_pre_kernel,
        grid=(grid,),
        in_specs=[
            pl.BlockSpec((blk, h.shape[1]), lambda i: (i, 0)),
            pl.BlockSpec((blk, p.shape[1]), lambda i: (i, 0)),
            pl.BlockSpec(wh.shape, lambda i: (0, 0)),
            pl.BlockSpec(wp.shape, lambda i: (0, 0)),
        ],
        out_specs=[pl.BlockSpec((blk, 128), lambda i: (i, 0))] * 4,
        out_shape=[out, out, out, out],
    )(h, p, wh, wp)


def _edge_mlp_kernel(uh_ref, up_ref, e_ref, weh_ref, wep_ref, b1h_ref, b1p_ref,
                     w2h_ref, w2p_ref, b2h_ref, b2p_ref, mh_ref, mp_ref):
    e = e_ref[...]
    th = uh_ref[...] + jnp.dot(e, weh_ref[...], preferred_element_type=jnp.float32) + b1h_ref[...]
    th = jax.nn.relu(th)
    mh_ref[...] = jnp.dot(th, w2h_ref[...], preferred_element_type=jnp.float32) + b2h_ref[...]
    tp = up_ref[...] + jnp.dot(e, wep_ref[...], preferred_element_type=jnp.float32) + b1p_ref[...]
    tp = jnp.tanh(tp)
    mp_ref[...] = jnp.dot(tp, w2p_ref[...], preferred_element_type=jnp.float32) + b2p_ref[...]


def _edge_mlp(uh, up, e, weh, wep, b1h, b1p, w2h, w2p, b2h, b2p, blk=2000):
    ne = e.shape[0]
    grid = ne // blk
    out = jax.ShapeDtypeStruct((ne, 128), jnp.float32)
    full = lambda a: pl.BlockSpec(a.shape, lambda i: (0,) * a.ndim)
    return pl.pallas_call(
        _edge_mlp_kernel,
        grid=(grid,),
        in_specs=[
            pl.BlockSpec((blk, 128), lambda i: (i, 0)),
            pl.BlockSpec((blk, 128), lambda i: (i, 0)),
            pl.BlockSpec((blk, e.shape[1]), lambda i: (i, 0)),
            full(weh), full(wep), full(b1h), full(b1p),
            full(w2h), full(w2p), full(b2h), full(b2p),
        ],
        out_specs=[pl.BlockSpec((blk, 128), lambda i: (i, 0))] * 2,
        out_shape=[out, out],
    )(uh, up, e, weh, wep, b1h, b1p, w2h, w2p, b2h, b2p)


def _post_kernel(h_ref, p_ref, snorm_ref, deg_ref,
                 sh_ref, qh_ref, xh_ref, nh_ref,
                 sp_ref, qp_ref, xp_ref, np_ref,
                 w1hh_ref, w1ha_ref, w1hb_ref, w1hc_ref, b1h_ref, w2h_ref, b2h_ref,
                 w1pp_ref, w1pa_ref, w1pb_ref, w1pc_ref, b1p_ref, w2p_ref, b2p_ref,
                 hpre_ref, pout_ref, psum_ref, psq_ref):
    deg = deg_ref[...]
    degc = jnp.maximum(deg, 1.0)
    present = deg > 0.0

    def agg_of(s, q, mx, mn):
        mean = s / degc
        mx = jnp.where(present, mx, 0.0)
        mn = jnp.where(present, mn, 0.0)
        var = jax.nn.relu(q / degc - mean * mean)
        std = jnp.sqrt(var + 1e-5)
        return jnp.concatenate([mean, mx, mn, std], axis=1)

    agg_h = agg_of(sh_ref[...], qh_ref[...], xh_ref[...], nh_ref[...])
    agg_p = agg_of(sp_ref[...], qp_ref[...], xp_ref[...], np_ref[...])
    logd = jnp.log(degc[:, :1] + 1.0)
    s1 = logd / _AVG_D_LOG
    s2 = _AVG_D_LOG / logd

    def post(xself, w_self, agg, wa, wb, wc, b1, w2, b2, act):
        y = jnp.dot(xself, w_self[...], preferred_element_type=jnp.float32)
        y += jnp.dot(agg, wa[...], preferred_element_type=jnp.float32)
        y += s1 * jnp.dot(agg, wb[...], preferred_element_type=jnp.float32)
        y += s2 * jnp.dot(agg, wc[...], preferred_element_type=jnp.float32)
        y = act(y + b1[...])
        return jnp.dot(y, w2[...], preferred_element_type=jnp.float32) + b2[...]

    hp = post(h_ref[...], w1hh_ref, agg_h, w1ha_ref, w1hb_ref, w1hc_ref,
              b1h_ref, w2h_ref, b2h_ref, jax.nn.relu)
    hp = hp * snorm_ref[...]
    hpre_ref[...] = hp
    pout_ref[...] = post(p_ref[...], w1pp_ref, agg_p, w1pa_ref, w1pb_ref, w1pc_ref,
                         b1p_ref, w2p_ref, b2p_ref, jnp.tanh)
    psum_ref[...] = jnp.sum(hp, axis=0, keepdims=True)[None]
    psq_ref[...] = jnp.sum(hp * hp, axis=0, keepdims=True)[None]


def _post(h, p, snorm, deg, aggs, wpost, blk=1000):
    n = h.shape[0]
    grid = n // blk
    sh, qh, xh, nh, sp, qp, xp, np_ = aggs
    full = lambda a: pl.BlockSpec(a.shape, lambda i: (0,) * a.ndim)
    row = lambda w: pl.BlockSpec((blk, w), lambda i: (i, 0))
    return pl.pallas_call(
        _post_kernel,
        grid=(grid,),
        in_specs=[row(h.shape[1]), row(p.shape[1]), row(1), row(128)]
                 + [row(128)] * 8
                 + [full(w) for w in wpost],
        out_specs=[row(128), row(128),
                   pl.BlockSpec((1, 1, 128), lambda i: (i, 0, 0)),
                   pl.BlockSpec((1, 1, 128), lambda i: (i, 0, 0))],
        out_shape=[jax.ShapeDtypeStruct((n, 128), jnp.float32),
                   jax.ShapeDtypeStruct((n, 128), jnp.float32),
                   jax.ShapeDtypeStruct((grid, 1, 128), jnp.float32),
                   jax.ShapeDtypeStruct((grid, 1, 128), jnp.float32)],
    )(h, p, snorm, deg, sh, qh, xh, nh, sp, qp, xp, np_, *wpost)


def _bn_kernel(hpre_ref, psum_ref, psq_ref, g_ref, b_ref, o_ref, *, n_total):
    mu = jnp.sum(psum_ref[...], axis=(0, 1), keepdims=False)[None] / n_total
    var = jnp.sum(psq_ref[...], axis=(0, 1), keepdims=False)[None] / n_total - mu * mu
    inv = jax.lax.rsqrt(var + 1e-5)
    o_ref[...] = (hpre_ref[...] - mu) * inv * g_ref[...] + b_ref[...]


def _bn(hpre, psum, psq, gamma, beta, blk=1000):
    n = hpre.shape[0]
    grid = n // blk
    full = lambda a: pl.BlockSpec(a.shape, lambda i: (0,) * a.ndim)
    return pl.pallas_call(
        functools.partial(_bn_kernel, n_total=float(n)),
        grid=(grid,),
        in_specs=[pl.BlockSpec((blk, 128), lambda i: (i, 0)),
                  full(psum), full(psq), full(gamma), full(beta)],
        out_specs=pl.BlockSpec((blk, 128), lambda i: (i, 0)),
        out_shape=jax.ShapeDtypeStruct((n, 128), jnp.float32),
    )(hpre, psum, psq, gamma, beta)


def _sc_gather(ah, bh, ap, bp, src, dst):
    n_e = src.shape[0]
    per_w = n_e // _NW
    w_blk = 200
    n_win = per_w // w_blk
    mesh = plsc.VectorSubcoreMesh(core_axis_name="c", subcore_axis_name="s")
    out = jax.ShapeDtypeStruct((n_e, 128), jnp.float32)

    @functools.partial(
        pl.kernel, mesh=mesh,
        out_type=[out, out],
        scratch_types=[
            pltpu.VMEM((w_blk,), jnp.int32),
            pltpu.VMEM((w_blk,), jnp.int32),
            pltpu.VMEM((w_blk, 128), jnp.float32),
            pltpu.VMEM((w_blk, 128), jnp.float32),
            pltpu.SemaphoreType.DMA,
        ],
    )
    def k(ah_h, bh_h, ap_h, bp_h, src_h, dst_h, uh_h, up_h,
          si_v, di_v, ra_v, rb_v, sem):
        wid = lax.axis_index("s") * _NC + lax.axis_index("c")
        base0 = wid * per_w

        def do_path(a_h, b_h, u_h):
            def body(w, _):
                base = base0 + w * w_blk
                pltpu.sync_copy(src_h.at[pl.ds(base, w_blk)], si_v)
                pltpu.sync_copy(dst_h.at[pl.ds(base, w_blk)], di_v)
                ca = pltpu.async_copy(a_h.at[si_v], ra_v, sem)
                cb = pltpu.async_copy(b_h.at[di_v], rb_v, sem)
                ca.wait()
                cb.wait()

                def add_row(i, _):
                    for j in range(8):
                        sl = pl.ds(j * 16, 16)
                        ra_v[i, sl] = ra_v[i, sl] + rb_v[i, sl]
                    return 0
                lax.fori_loop(0, w_blk, add_row, 0)
                pltpu.sync_copy(ra_v, u_h.at[pl.ds(base, w_blk)])
                return 0
            lax.fori_loop(0, n_win, body, 0)

        do_path(ah_h, bh_h, uh_h)
        do_path(ap_h, bp_h, up_h)

    return k(ah, bh, ap, bp, src, dst)


def _sc_sum_sq(mh, mp, dst, n_pad):
    n_e = dst.shape[0]
    per_t = n_e // _NS
    w_blk = 200
    n_win = per_t // w_blk
    zr = 64
    mesh = plsc.VectorSubcoreMesh(core_axis_name="c", subcore_axis_name="s")
    out = jax.ShapeDtypeStruct((n_pad, 128), jnp.float32)

    @functools.partial(
        pl.kernel, mesh=mesh,
        out_type=[out] * 4,
        scratch_types=[
            pltpu.VMEM((w_blk,), jnp.int32),
            pltpu.VMEM((w_blk, 128), jnp.float32),
            pltpu.VMEM((zr, 128), jnp.float32),
            pltpu.VMEM_SHARED((n_pad, 128), jnp.float32),
            pltpu.SemaphoreType.DMA,
        ],
    )
    def k(mh_h, mp_h, dst_h, sh_h, qh_h, sp_h, qp_h,
          di_v, m_v, z_v, acc_s, sem):
        cid = lax.axis_index("c")
        sid = lax.axis_index("s")

        def zrow(i, _):
            for j in range(8):
                z_v[i, pl.ds(j * 16, 16)] = jnp.zeros((16,), jnp.float32)
            return 0
        lax.fori_loop(0, zr, zrow, 0)

        def do_path(m_hbm, s_out, q_out):
            def zchunk(q, _):
                pltpu.sync_copy(z_v, acc_s.at[pl.ds(sid * 640 + q * zr, zr)])
                return 0
            lax.fori_loop(0, 10, zchunk, 0)
            plsc.subcore_barrier()

            def body(w, _):
                base = sid * per_t + w * w_blk
                pltpu.sync_copy(dst_h.at[pl.ds(base, w_blk)], di_v)
                pltpu.sync_copy(m_hbm.at[pl.ds(base, w_blk)], m_v)

                @pl.when(cid == 1)
                def _():
                    def sq(i, _):
                        for j in range(8):
                            sl = pl.ds(j * 16, 16)
                            v = m_v[i, sl]
                            m_v[i, sl] = v * v
                        return 0
                    lax.fori_loop(0, w_blk, sq, 0)

                pltpu.sync_copy(m_v, acc_s.at[di_v], add=True)
                return 0
            lax.fori_loop(0, n_win, body, 0)
            plsc.subcore_barrier()
            rows = pl.ds(sid * 640, 640)

            @pl.when(cid == 0)
            def _():
                pltpu.sync_copy(acc_s.at[rows], s_out.at[rows])

            @pl.when(cid == 1)
            def _():
                pltpu.sync_copy(acc_s.at[rows], q_out.at[rows])

            plsc.subcore_barrier()

        do_path(mh_h, sh_h, qh_h)
        do_path(mp_h, sp_h, qp_h)

    return k(mh, mp, dst)


def _sc_max_min_deg(mh, mp, dst, n_pad):
    n_e = dst.shape[0]
    rs = 320
    wd = 2000
    n_win = n_e // wd
    mesh = plsc.VectorSubcoreMesh(core_axis_name="c", subcore_axis_name="s")
    out = jax.ShapeDtypeStruct((n_pad, 128), jnp.float32)
    dout = jax.ShapeDtypeStruct((n_pad, 16), jnp.float32)

    @functools.partial(
        pl.kernel, mesh=mesh,
        out_type=[out, out, out, out, dout],
        compiler_params=pltpu.CompilerParams(needs_layout_passes=False),
        scratch_types=[
            pltpu.VMEM((wd,), jnp.int32),
            pltpu.VMEM((wd + 16,), jnp.int32),
            pltpu.VMEM((wd + 16,), jnp.int32),
            pltpu.VMEM((16, 128), jnp.float32),
            pltpu.VMEM((rs, 128), jnp.float32),
            pltpu.VMEM((rs, 128), jnp.float32),
            pltpu.VMEM((rs, 16), jnp.float32),
            pltpu.SemaphoreType.DMA,
        ],
    )
    def k(mh_h, mp_h, dst_h, xh_h, nh_h, xp_h, np_h, deg_h,
          dw_v, hd_v, hi_v, mrow_v, amx_v, amn_v, deg_v, sem):
        wid = lax.axis_index("s") * _NC + lax.axis_index("c")
        lo = wid * rs

        def izero(kk, _):
            hi_v[pl.ds(kk * 16, 16)] = jnp.zeros((16,), jnp.int32)
            return 0
        lax.fori_loop(0, (wd + 16) // 16, izero, 0)

        def dzero(i, _):
            deg_v[i, :] = jnp.zeros((16,), jnp.float32)
            return 0
        lax.fori_loop(0, rs, dzero, 0)

        def do_pass(m_hbm, x_out, n_out, count_deg):
            big = jnp.full((16,), 3.0e38, jnp.float32)

            def ainit(i, _):
                for u in range(8):
                    sl = pl.ds(u * 16, 16)
                    amx_v[i, sl] = -big
                    amn_v[i, sl] = big
                return 0
            lax.fori_loop(0, rs, ainit, 0)

            def win_body(w, _):
                pltpu.sync_copy(dst_h.at[pl.ds(w * wd, wd)], dw_v)

                def scan(kk, nhits):
                    d = dw_v[pl.ds(kk * 16, 16)]
                    dl = d - lo
                    m = (dl >= 0) & (dl < rs)
                    cnt = plsc.all_reduce_population_count(m)[0]
                    ev = w * wd + kk * 16 + lax.iota(jnp.int32, 16)
                    pos = nhits + plsc.cumsum(m.astype(jnp.int32)) - 1
                    plsc.store_scatter(hd_v, [pos], dl, mask=m)
                    plsc.store_scatter(hi_v, [pos], ev, mask=m)
                    return nhits + cnt
                nhits = lax.fori_loop(0, wd // 16, scan, jnp.int32(0))

                one16 = jnp.ones((16,), jnp.float32)

                def chunk(j, _):
                    iv = hi_v[pl.ds(j * 16, 16)]
                    dlv = hd_v[pl.ds(j * 16, 16)]
                    pltpu.async_copy(m_hbm.at[iv], mrow_v, sem).wait()
                    for t in range(16):
                        @pl.when(j * 16 + t < nhits)
                        def _(t=t):
                            dl = dlv[t]
                            for u in range(8):
                                sl = pl.ds(u * 16, 16)
                                v = mrow_v[t, sl]
                                amx_v[dl, sl] = jnp.maximum(amx_v[dl, sl], v)
                                amn_v[dl, sl] = jnp.minimum(amn_v[dl, sl], v)
                            if count_deg:
                                deg_v[dl, :] = deg_v[dl, :] + one16
                    return 0
                lax.fori_loop(0, (nhits + 15) // 16, chunk, 0)
                return 0
            lax.fori_loop(0, n_win, win_body, 0)
            rows = pl.ds(lo, rs)
            pltpu.sync_copy(amx_v, x_out.at[rows])
            pltpu.sync_copy(amn_v, n_out.at[rows])
            if count_deg:
                pltpu.sync_copy(deg_v, deg_h.at[rows])

        do_pass(mh_h, xh_h, nh_h, True)
        do_pass(mp_h, xp_h, np_h, False)

    return k(mh, mp, dst)


def kernel(h, p, e, snorm_n, edge_index,
           pre_h_W1, pre_h_b1, pre_h_W2, pre_h_b2,
           pre_p_W1, pre_p_b1, pre_p_W2, pre_p_b2,
           post_h_W1, post_h_b1, post_h_W2, post_h_b2,
           post_p_W1, post_p_b1, post_p_W2, post_p_b2,
           bn_gamma, bn_beta):
    n = h.shape[0]
    ne = e.shape[0]
    src = edge_index[0]
    dst = edge_index[1]

    f = 128
    wh_node = jnp.concatenate([pre_h_W1[:2 * f], pre_h_W1[2 * f:4 * f]], axis=1)
    wp_node = jnp.concatenate([pre_p_W1[:f], pre_p_W1[f:2 * f]], axis=1)
    weh = pre_h_W1[4 * f:]
    wep = pre_p_W1[2 * f:]
    b1h = pre_h_b1.reshape(1, -1)
    b1p = pre_p_b1.reshape(1, -1)
    b2h = pre_h_b2.reshape(1, -1)
    b2p = pre_p_b2.reshape(1, -1)

    ah, bh, ap, bp = _node_pre(h, p, wh_node, wp_node)

    uh, up = _sc_gather(ah, bh, ap, bp, src, dst)

    mh, mp = _edge_mlp(uh, up, e, weh, wep, b1h, b1p,
                       pre_h_W2, pre_p_W2, b2h, b2p)

    shp, qhp, spp, qpp = _sc_sum_sq(mh, mp, dst, 10240)
    sh, qh, sp, qp = shp[:n], qhp[:n], spp[:n], qpp[:n]

    xhp, nhp, xpp, npp, degp = _sc_max_min_deg(mh, mp, dst, 10240)
    xh, nh, xp, np_ = xhp[:n], nhp[:n], xpp[:n], npp[:n]
    deg = degp[:n, 0]
    degb = jnp.broadcast_to(deg[:, None], (n, 128))

    wpost = [post_h_W1[:2 * f], post_h_W1[2 * f:6 * f],
             post_h_W1[6 * f:10 * f], post_h_W1[10 * f:14 * f],
             post_h_b1.reshape(1, -1), post_h_W2, post_h_b2.reshape(1, -1),
             post_p_W1[:f], post_p_W1[f:5 * f],
             post_p_W1[5 * f:9 * f], post_p_W1[9 * f:13 * f],
             post_p_b1.reshape(1, -1), post_p_W2, post_p_b2.reshape(1, -1)]

    hpre, pout, psum, psq = _post(h, p, snorm_n, degb,
                                  (sh, qh, xh, nh, sp, qp, xp, np_), wpost)
    hout = _bn(hpre, psum, psq, bn_gamma.reshape(1, -1), bn_beta.reshape(1, -1))
    return (hout, pout)


# speedup vs baseline: 2.4045x; 1.0956x over previous
import functools

import jax
import jax.numpy as jnp
from jax import lax
from jax.experimental import pallas as pl
from jax.experimental.pallas import tpu as pltpu
from jax.experimental.pallas import tpu_sc as plsc

_NC, _NS = 2, 16
_NW = _NC * _NS

_AVG_D_LOG = 3.4965075614664802


def _node_pre_kernel(h_ref, p_ref, wh_ref, wp_ref, ah_ref, bh_ref, ap_ref, bp_ref):
    abh = jnp.dot(h_ref[...], wh_ref[...], preferred_element_type=jnp.float32)
    abp = jnp.dot(p_ref[...], wp_ref[...], preferred_element_type=jnp.float32)
    ah_ref[...] = abh[:, :128]
    bh_ref[...] = abh[:, 128:]
    ap_ref[...] = abp[:, :128]
    bp_ref[...] = abp[:, 128:]


def _node_pre(h, p, wh, wp, blk=1000):
    n = h.shape[0]
    grid = n // blk
    out = jax.ShapeDtypeStruct((n, 128), jnp.float32)
    return pl.pallas_call(
        _node_pre_kernel,
        grid=(grid,),
        in_specs=[
            pl.BlockSpec((blk, h.shape[1]), lambda i: (i, 0)),
            pl.BlockSpec((blk, p.shape[1]), lambda i: (i, 0)),
            pl.BlockSpec(wh.shape, lambda i: (0, 0)),
            pl.BlockSpec(wp.shape, lambda i: (0, 0)),
        ],
        out_specs=[pl.BlockSpec((blk, 128), lambda i: (i, 0))] * 4,
        out_shape=[out, out, out, out],
    )(h, p, wh, wp)


def _edge_mlp_kernel(uh_ref, up_ref, e_ref, weh_ref, wep_ref, b1h_ref, b1p_ref,
                     w2h_ref, w2p_ref, b2h_ref, b2p_ref, mh_ref, mp_ref):
    e = e_ref[...]
    th = uh_ref[...] + jnp.dot(e, weh_ref[...], preferred_element_type=jnp.float32) + b1h_ref[...]
    th = jax.nn.relu(th)
    mh_ref[...] = jnp.dot(th, w2h_ref[...], preferred_element_type=jnp.float32) + b2h_ref[...]
    tp = up_ref[...] + jnp.dot(e, wep_ref[...], preferred_element_type=jnp.float32) + b1p_ref[...]
    tp = jnp.tanh(tp)
    mp_ref[...] = jnp.dot(tp, w2p_ref[...], preferred_element_type=jnp.float32) + b2p_ref[...]


def _edge_mlp(uh, up, e, weh, wep, b1h, b1p, w2h, w2p, b2h, b2p, blk=2000):
    ne = e.shape[0]
    grid = ne // blk
    out = jax.ShapeDtypeStruct((ne, 128), jnp.float32)
    full = lambda a: pl.BlockSpec(a.shape, lambda i: (0,) * a.ndim)
    return pl.pallas_call(
        _edge_mlp_kernel,
        grid=(grid,),
        in_specs=[
            pl.BlockSpec((blk, 128), lambda i: (i, 0)),
            pl.BlockSpec((blk, 128), lambda i: (i, 0)),
            pl.BlockSpec((blk, e.shape[1]), lambda i: (i, 0)),
            full(weh), full(wep), full(b1h), full(b1p),
            full(w2h), full(w2p), full(b2h), full(b2p),
        ],
        out_specs=[pl.BlockSpec((blk, 128), lambda i: (i, 0))] * 2,
        out_shape=[out, out],
    )(uh, up, e, weh, wep, b1h, b1p, w2h, w2p, b2h, b2p)


def _post_kernel(h_ref, p_ref, snorm_ref, deg_ref,
                 sh_ref, qh_ref, xh_ref, nh_ref,
                 sp_ref, qp_ref, xp_ref, np_ref,
                 w1hh_ref, w1ha_ref, w1hb_ref, w1hc_ref, b1h_ref, w2h_ref, b2h_ref,
                 w1pp_ref, w1pa_ref, w1pb_ref, w1pc_ref, b1p_ref, w2p_ref, b2p_ref,
                 hpre_ref, pout_ref, psum_ref, psq_ref):
    deg = deg_ref[...]
    degc = jnp.maximum(deg, 1.0)
    present = deg > 0.0

    def agg_of(s, q, mx, mn):
        mean = s / degc
        mx = jnp.where(present, mx, 0.0)
        mn = jnp.where(present, mn, 0.0)
        var = jax.nn.relu(q / degc - mean * mean)
        std = jnp.sqrt(var + 1e-5)
        return jnp.concatenate([mean, mx, mn, std], axis=1)

    agg_h = agg_of(sh_ref[...], qh_ref[...], xh_ref[...], nh_ref[...])
    agg_p = agg_of(sp_ref[...], qp_ref[...], xp_ref[...], np_ref[...])
    logd = jnp.log(degc[:, :1] + 1.0)
    s1 = logd / _AVG_D_LOG
    s2 = _AVG_D_LOG / logd

    def post(xself, w_self, agg, wa, wb, wc, b1, w2, b2, act):
        y = jnp.dot(xself, w_self[...], preferred_element_type=jnp.float32)
        y += jnp.dot(agg, wa[...], preferred_element_type=jnp.float32)
        y += s1 * jnp.dot(agg, wb[...], preferred_element_type=jnp.float32)
        y += s2 * jnp.dot(agg, wc[...], preferred_element_type=jnp.float32)
        y = act(y + b1[...])
        return jnp.dot(y, w2[...], preferred_element_type=jnp.float32) + b2[...]

    hp = post(h_ref[...], w1hh_ref, agg_h, w1ha_ref, w1hb_ref, w1hc_ref,
              b1h_ref, w2h_ref, b2h_ref, jax.nn.relu)
    hp = hp * snorm_ref[...]
    hpre_ref[...] = hp
    pout_ref[...] = post(p_ref[...], w1pp_ref, agg_p, w1pa_ref, w1pb_ref, w1pc_ref,
                         b1p_ref, w2p_ref, b2p_ref, jnp.tanh)
    psum_ref[...] = jnp.sum(hp, axis=0, keepdims=True)[None]
    psq_ref[...] = jnp.sum(hp * hp, axis=0, keepdims=True)[None]


def _post(h, p, snorm, deg, aggs, wpost, blk=1000):
    n = h.shape[0]
    grid = n // blk
    sh, qh, xh, nh, sp, qp, xp, np_ = aggs
    full = lambda a: pl.BlockSpec(a.shape, lambda i: (0,) * a.ndim)
    row = lambda w: pl.BlockSpec((blk, w), lambda i: (i, 0))
    return pl.pallas_call(
        _post_kernel,
        grid=(grid,),
        in_specs=[row(h.shape[1]), row(p.shape[1]), row(1), row(128)]
                 + [row(128)] * 8
                 + [full(w) for w in wpost],
        out_specs=[row(128), row(128),
                   pl.BlockSpec((1, 1, 128), lambda i: (i, 0, 0)),
                   pl.BlockSpec((1, 1, 128), lambda i: (i, 0, 0))],
        out_shape=[jax.ShapeDtypeStruct((n, 128), jnp.float32),
                   jax.ShapeDtypeStruct((n, 128), jnp.float32),
                   jax.ShapeDtypeStruct((grid, 1, 128), jnp.float32),
                   jax.ShapeDtypeStruct((grid, 1, 128), jnp.float32)],
    )(h, p, snorm, deg, sh, qh, xh, nh, sp, qp, xp, np_, *wpost)


def _bn_kernel(hpre_ref, psum_ref, psq_ref, g_ref, b_ref, o_ref, *, n_total):
    mu = jnp.sum(psum_ref[...], axis=(0, 1), keepdims=False)[None] / n_total
    var = jnp.sum(psq_ref[...], axis=(0, 1), keepdims=False)[None] / n_total - mu * mu
    inv = jax.lax.rsqrt(var + 1e-5)
    o_ref[...] = (hpre_ref[...] - mu) * inv * g_ref[...] + b_ref[...]


def _bn(hpre, psum, psq, gamma, beta, blk=1000):
    n = hpre.shape[0]
    grid = n // blk
    full = lambda a: pl.BlockSpec(a.shape, lambda i: (0,) * a.ndim)
    return pl.pallas_call(
        functools.partial(_bn_kernel, n_total=float(n)),
        grid=(grid,),
        in_specs=[pl.BlockSpec((blk, 128), lambda i: (i, 0)),
                  full(psum), full(psq), full(gamma), full(beta)],
        out_specs=pl.BlockSpec((blk, 128), lambda i: (i, 0)),
        out_shape=jax.ShapeDtypeStruct((n, 128), jnp.float32),
    )(hpre, psum, psq, gamma, beta)


def _sc_gather(ah, bh, ap, bp, src, dst):
    n_e = src.shape[0]
    per_w = n_e // _NW
    w_blk = 200
    n_win = per_w // w_blk
    mesh = plsc.VectorSubcoreMesh(core_axis_name="c", subcore_axis_name="s")
    out = jax.ShapeDtypeStruct((n_e, 128), jnp.float32)

    @functools.partial(
        pl.kernel, mesh=mesh,
        out_type=[out, out],
        scratch_types=[
            pltpu.VMEM((w_blk,), jnp.int32),
            pltpu.VMEM((w_blk,), jnp.int32),
            pltpu.VMEM((w_blk, 128), jnp.float32),
            pltpu.VMEM((w_blk, 128), jnp.float32),
            pltpu.SemaphoreType.DMA,
        ],
    )
    def k(ah_h, bh_h, ap_h, bp_h, src_h, dst_h, uh_h, up_h,
          si_v, di_v, ra_v, rb_v, sem):
        wid = lax.axis_index("s") * _NC + lax.axis_index("c")
        base0 = wid * per_w

        def do_path(a_h, b_h, u_h):
            def body(w, _):
                base = base0 + w * w_blk
                pltpu.sync_copy(src_h.at[pl.ds(base, w_blk)], si_v)
                pltpu.sync_copy(dst_h.at[pl.ds(base, w_blk)], di_v)
                ca = pltpu.async_copy(a_h.at[si_v], ra_v, sem)
                cb = pltpu.async_copy(b_h.at[di_v], rb_v, sem)
                ca.wait()
                cb.wait()

                def add_row(i, _):
                    for j in range(8):
                        sl = pl.ds(j * 16, 16)
                        ra_v[i, sl] = ra_v[i, sl] + rb_v[i, sl]
                    return 0
                lax.fori_loop(0, w_blk, add_row, 0)
                pltpu.sync_copy(ra_v, u_h.at[pl.ds(base, w_blk)])
                return 0
            lax.fori_loop(0, n_win, body, 0)

        do_path(ah_h, bh_h, uh_h)
        do_path(ap_h, bp_h, up_h)

    return k(ah, bh, ap, bp, src, dst)


def _sc_sum_sq(mh, mp, dst, n_pad):
    n_e = dst.shape[0]
    per_t = n_e // _NS
    w_blk = 200
    n_win = per_t // w_blk
    zr = 64
    mesh = plsc.VectorSubcoreMesh(core_axis_name="c", subcore_axis_name="s")
    out = jax.ShapeDtypeStruct((n_pad, 128), jnp.float32)

    @functools.partial(
        pl.kernel, mesh=mesh,
        out_type=[out] * 4,
        scratch_types=[
            pltpu.VMEM((w_blk,), jnp.int32),
            pltpu.VMEM((w_blk, 128), jnp.float32),
            pltpu.VMEM((zr, 128), jnp.float32),
            pltpu.VMEM_SHARED((n_pad, 128), jnp.float32),
            pltpu.SemaphoreType.DMA,
        ],
    )
    def k(mh_h, mp_h, dst_h, sh_h, qh_h, sp_h, qp_h,
          di_v, m_v, z_v, acc_s, sem):
        cid = lax.axis_index("c")
        sid = lax.axis_index("s")

        def zrow(i, _):
            for j in range(8):
                z_v[i, pl.ds(j * 16, 16)] = jnp.zeros((16,), jnp.float32)
            return 0
        lax.fori_loop(0, zr, zrow, 0)

        def do_path(m_hbm, s_out, q_out):
            def zchunk(q, _):
                pltpu.sync_copy(z_v, acc_s.at[pl.ds(sid * 640 + q * zr, zr)])
                return 0
            lax.fori_loop(0, 10, zchunk, 0)
            plsc.subcore_barrier()

            def body(w, _):
                base = sid * per_t + w * w_blk
                pltpu.sync_copy(dst_h.at[pl.ds(base, w_blk)], di_v)
                pltpu.sync_copy(m_hbm.at[pl.ds(base, w_blk)], m_v)

                @pl.when(cid == 1)
                def _():
                    def sq(i, _):
                        for j in range(8):
                            sl = pl.ds(j * 16, 16)
                            v = m_v[i, sl]
                            m_v[i, sl] = v * v
                        return 0
                    lax.fori_loop(0, w_blk, sq, 0)

                pltpu.sync_copy(m_v, acc_s.at[di_v], add=True)
                return 0
            lax.fori_loop(0, n_win, body, 0)
            plsc.subcore_barrier()
            rows = pl.ds(sid * 640, 640)

            @pl.when(cid == 0)
            def _():
                pltpu.sync_copy(acc_s.at[rows], s_out.at[rows])

            @pl.when(cid == 1)
            def _():
                pltpu.sync_copy(acc_s.at[rows], q_out.at[rows])

            plsc.subcore_barrier()

        do_path(mh_h, sh_h, qh_h)
        do_path(mp_h, sp_h, qp_h)

    return k(mh, mp, dst)


def _sc_max_min_deg(mh, mp, dst, n_pad):
    n_e = dst.shape[0]
    rs = 320
    wd = 2000
    n_win = n_e // wd
    mesh = plsc.VectorSubcoreMesh(core_axis_name="c", subcore_axis_name="s")
    out = jax.ShapeDtypeStruct((n_pad, 128), jnp.float32)
    dout = jax.ShapeDtypeStruct((n_pad * 16,), jnp.float32)

    rsp = rs + 8
    @functools.partial(
        pl.kernel, mesh=mesh,
        out_type=[out, out, out, out, dout],
        compiler_params=pltpu.CompilerParams(needs_layout_passes=False),
        scratch_types=[
            pltpu.VMEM((wd,), jnp.int32),
            pltpu.VMEM((wd + 16,), jnp.int32),
            pltpu.VMEM((wd + 16,), jnp.int32),
            pltpu.VMEM((16, 128), jnp.float32),
            pltpu.VMEM((16, 128), jnp.float32),
            pltpu.VMEM((rsp, 128), jnp.float32),
            pltpu.VMEM((rsp, 128), jnp.float32),
            pltpu.VMEM((rsp * 16,), jnp.float32),
            pltpu.SemaphoreType.DMA,
        ],
    )
    def k(mh_h, mp_h, dst_h, xh_h, nh_h, xp_h, np_h, deg_h,
          dw_v, hd_v, hi_v, mrow_a, mrow_b, amx_v, amn_v, deg_v, sem):
        wid = lax.axis_index("s") * _NC + lax.axis_index("c")
        lo = wid * rs

        def izero(kk, _):
            hi_v[pl.ds(kk * 16, 16)] = jnp.zeros((16,), jnp.int32)
            return 0
        lax.fori_loop(0, (wd + 16) // 16, izero, 0)

        def dzero(i, _):
            deg_v[pl.ds(i * 16, 16)] = jnp.zeros((16,), jnp.float32)
            return 0
        lax.fori_loop(0, rsp, dzero, 0)

        def do_pass(m_hbm, x_out, n_out, count_deg):
            big = jnp.full((16,), 3.0e38, jnp.float32)
            one16 = jnp.ones((16,), jnp.float32)
            pad16 = jnp.full((16,), rs, jnp.int32)

            def ainit(i, _):
                for u in range(8):
                    sl = pl.ds(u * 16, 16)
                    amx_v[i, sl] = -big
                    amn_v[i, sl] = big
                return 0
            lax.fori_loop(0, rsp, ainit, 0)

            def process(buf, dlv, count_deg):
                for t in range(16):
                    dl = dlv[t]
                    for u in range(8):
                        sl = pl.ds(u * 16, 16)
                        v = buf[t, sl]
                        amx_v[dl, sl] = jnp.maximum(amx_v[dl, sl], v)
                        amn_v[dl, sl] = jnp.minimum(amn_v[dl, sl], v)
                    if count_deg:
                        ds16 = pl.ds(dl * 16, 16)
                        deg_v[ds16] = deg_v[ds16] + one16

            def win_body(w, _):
                pltpu.sync_copy(dst_h.at[pl.ds(w * wd, wd)], dw_v)

                def scan(kk, nhits):
                    d = dw_v[pl.ds(kk * 16, 16)]
                    dl = d - lo
                    m = (dl >= 0) & (dl < rs)
                    cnt = plsc.all_reduce_population_count(m)[0]
                    ev = w * wd + kk * 16 + lax.iota(jnp.int32, 16)
                    plsc.store_compressed(hd_v.at[pl.ds(nhits, 16)], dl, mask=m)
                    plsc.store_compressed(hi_v.at[pl.ds(nhits, 16)], ev, mask=m)
                    return nhits + cnt
                nhits = lax.fori_loop(0, wd // 16, scan, jnp.int32(0))
                hd_v[pl.ds(nhits, 16)] = pad16
                nch = (nhits + 15) // 16

                def pair(q, _):
                    j0 = 2 * q
                    j1 = j0 + 1
                    iv0 = hi_v[pl.ds(j0 * 16, 16)]
                    iv1 = hi_v[pl.ds(j1 * 16, 16)]
                    c0 = pltpu.make_async_copy(m_hbm.at[iv0], mrow_a, sem)
                    c0.start()

                    @pl.when(j1 < nch)
                    def _():
                        pltpu.make_async_copy(m_hbm.at[iv1], mrow_b, sem).start()

                    c0.wait()
                    process(mrow_a, hd_v[pl.ds(j0 * 16, 16)], count_deg)

                    @pl.when(j1 < nch)
                    def _():
                        pltpu.make_async_copy(m_hbm.at[iv1], mrow_b, sem).wait()
                        process(mrow_b, hd_v[pl.ds(j1 * 16, 16)], count_deg)
                    return 0
                lax.fori_loop(0, (nch + 1) // 2, pair, 0)
                return 0
            lax.fori_loop(0, n_win, win_body, 0)
            rows = pl.ds(lo, rs)
            pltpu.sync_copy(amx_v.at[pl.ds(0, rs)], x_out.at[rows])
            pltpu.sync_copy(amn_v.at[pl.ds(0, rs)], n_out.at[rows])
            if count_deg:
                pltpu.sync_copy(deg_v.at[pl.ds(0, rs * 16)],
                                deg_h.at[pl.ds(lo * 16, rs * 16)])

        do_pass(mh_h, xh_h, nh_h, True)
        do_pass(mp_h, xp_h, np_h, False)

    return k(mh, mp, dst)


def kernel(h, p, e, snorm_n, edge_index,
           pre_h_W1, pre_h_b1, pre_h_W2, pre_h_b2,
           pre_p_W1, pre_p_b1, pre_p_W2, pre_p_b2,
           post_h_W1, post_h_b1, post_h_W2, post_h_b2,
           post_p_W1, post_p_b1, post_p_W2, post_p_b2,
           bn_gamma, bn_beta):
    n = h.shape[0]
    ne = e.shape[0]
    src = edge_index[0]
    dst = edge_index[1]

    f = 128
    wh_node = jnp.concatenate([pre_h_W1[:2 * f], pre_h_W1[2 * f:4 * f]], axis=1)
    wp_node = jnp.concatenate([pre_p_W1[:f], pre_p_W1[f:2 * f]], axis=1)
    weh = pre_h_W1[4 * f:]
    wep = pre_p_W1[2 * f:]
    b1h = pre_h_b1.reshape(1, -1)
    b1p = pre_p_b1.reshape(1, -1)
    b2h = pre_h_b2.reshape(1, -1)
    b2p = pre_p_b2.reshape(1, -1)

    ah, bh, ap, bp = _node_pre(h, p, wh_node, wp_node)

    uh, up = _sc_gather(ah, bh, ap, bp, src, dst)

    mh, mp = _edge_mlp(uh, up, e, weh, wep, b1h, b1p,
                       pre_h_W2, pre_p_W2, b2h, b2p)

    shp, qhp, spp, qpp = _sc_sum_sq(mh, mp, dst, 10240)
    sh, qh, sp, qp = shp[:n], qhp[:n], spp[:n], qpp[:n]

    xhp, nhp, xpp, npp, degp = _sc_max_min_deg(mh, mp, dst, 10240)
    xh, nh, xp, np_ = xhp[:n], nhp[:n], xpp[:n], npp[:n]
    deg = degp.reshape(10240, 16)[:n, 0]
    degb = jnp.broadcast_to(deg[:, None], (n, 128))

    wpost = [post_h_W1[:2 * f], post_h_W1[2 * f:6 * f],
             post_h_W1[6 * f:10 * f], post_h_W1[10 * f:14 * f],
             post_h_b1.reshape(1, -1), post_h_W2, post_h_b2.reshape(1, -1),
             post_p_W1[:f], post_p_W1[f:5 * f],
             post_p_W1[5 * f:9 * f], post_p_W1[9 * f:13 * f],
             post_p_b1.reshape(1, -1), post_p_W2, post_p_b2.reshape(1, -1)]

    hpre, pout, psum, psq = _post(h, p, snorm_n, degb,
                                  (sh, qh, xh, nh, sp, qp, xp, np_), wpost)
    hout = _bn(hpre, psum, psq, bn_gamma.reshape(1, -1), bn_beta.reshape(1, -1))
    return (hout, pout)
